# baseline (speedup 1.0000x reference)
import functools

import jax
import jax.numpy as jnp
from jax import lax
from jax.experimental import pallas as pl
from jax.experimental.pallas import tpu as pltpu
from jax.experimental.pallas import tpu_sc as plsc

_NC = 2
_NS = 16
_NW = _NC * _NS


def _gate_body(x_ref, wg_ref, gv_ref, gi_ref, ss_ref, ms_ref):
    i = pl.program_id(0)
    lg = lax.dot_general(x_ref[...].astype(jnp.bfloat16),
                         wg_ref[...].astype(jnp.bfloat16),
                         (((1,), (0,)), ((), ())),
                         preferred_element_type=jnp.float32)
    m = jnp.max(lg, axis=-1, keepdims=True)
    ex = jnp.exp(lg - m)
    p = ex / jnp.sum(ex, axis=-1, keepdims=True)
    e = p.shape[-1]
    i8 = lax.broadcasted_iota(jnp.int32, p.shape, 1)
    v1 = jnp.max(p, axis=-1, keepdims=True)
    a1 = jnp.min(jnp.where(p >= v1, i8, e), axis=-1)
    mask1 = i8 == a1[:, None]
    p2 = jnp.where(mask1, -1.0, p)
    v2 = jnp.max(p2, axis=-1, keepdims=True)
    a2 = jnp.min(jnp.where(p2 >= v2, i8, e), axis=-1)
    gv_ref[...] = jnp.concatenate([v1, v2], axis=-1)
    gi_ref[...] = jnp.concatenate([a1[:, None], a2[:, None]], axis=-1)

    @pl.when(i == 0)
    def _():
        ss_ref[...] = jnp.zeros_like(ss_ref)
        ms_ref[...] = jnp.zeros_like(ms_ref)

    ss_ref[...] += jnp.sum(p, axis=0, keepdims=True)
    ms_ref[...] += jnp.sum(mask1.astype(jnp.float32), axis=0, keepdims=True)


def _gate(x, wg):
    t, d = x.shape
    e = wg.shape[1]
    tb = min(512, t)
    grid = (t // tb,)
    return pl.pallas_call(
        _gate_body,
        grid=grid,
        in_specs=[
            pl.BlockSpec((tb, d), lambda i: (i, 0)),
            pl.BlockSpec((d, e), lambda i: (0, 0)),
        ],
        out_specs=[
            pl.BlockSpec((tb, 2), lambda i: (i, 0)),
            pl.BlockSpec((tb, 2), lambda i: (i, 0)),
            pl.BlockSpec((1, e), lambda i: (0, 0)),
            pl.BlockSpec((1, e), lambda i: (0, 0)),
        ],
        out_shape=[
            jax.ShapeDtypeStruct((t, 2), jnp.float32),
            jax.ShapeDtypeStruct((t, 2), jnp.int32),
            jax.ShapeDtypeStruct((1, e), jnp.float32),
            jax.ShapeDtypeStruct((1, e), jnp.float32),
        ],
        compiler_params=pltpu.CompilerParams(
            dimension_semantics=("arbitrary",)),
    )(x, wg)


def _ffn_body(nh, eob_ref, act_ref, xs_ref, w1_ref, b1_ref, w2_ref, b2_ref,
              g_ref, eo_ref):
    rb = pl.program_id(0)
    hb = pl.program_id(1)
    active = act_ref[rb] > 0

    @pl.when(active)
    def _():
        xb = xs_ref[...]
        w1b = w1_ref[0].astype(jnp.bfloat16)
        h = lax.dot_general(xb, w1b, (((1,), (0,)), ((), ())),
                            preferred_element_type=jnp.float32)
        h = jnp.maximum(h + b1_ref[0], 0.0).astype(jnp.bfloat16)
        pt = lax.dot_general(h, w2_ref[0].astype(jnp.bfloat16),
                             (((1,), (0,)), ((), ())),
                             preferred_element_type=jnp.float32)

        @pl.when(hb == 0)
        def _():
            eo_ref[...] = pt

        @pl.when(hb > 0)
        def _():
            eo_ref[...] += pt

        @pl.when(hb == nh - 1)
        def _():
            g = g_ref[0, 0, :][:, None]
            eo_ref[...] = g * (eo_ref[...] + b2_ref[0])


def _ffn(eob, act, xs, w1, b1, w2, b2, g3, blk, nh):
    npad, d = xs.shape
    e, _, h = w1.shape
    hblk = h // nh
    nb = npad // blk
    grid_spec = pltpu.PrefetchScalarGridSpec(
        num_scalar_prefetch=2,
        grid=(nb, nh),
        in_specs=[
            pl.BlockSpec((blk, d), lambda rb, hb, eob, act: (rb, 0)),
            pl.BlockSpec((1, d, hblk),
                         lambda rb, hb, eob, act: (eob[rb], 0, hb * act[rb])),
            pl.BlockSpec((1, 1, hblk),
                         lambda rb, hb, eob, act: (eob[rb], 0, hb * act[rb])),
            pl.BlockSpec((1, hblk, d),
                         lambda rb, hb, eob, act: (eob[rb], hb * act[rb], 0)),
            pl.BlockSpec((1, 1, d),
                         lambda rb, hb, eob, act: (eob[rb], 0, 0)),
            pl.BlockSpec((1, 1, blk), lambda rb, hb, eob, act: (rb, 0, 0)),
        ],
        out_specs=pl.BlockSpec((blk, d), lambda rb, hb, eob, act: (rb, 0)),
    )
    return pl.pallas_call(
        functools.partial(_ffn_body, nh),
        grid_spec=grid_spec,
        out_shape=jax.ShapeDtypeStruct((npad, d), jnp.float32),
        compiler_params=pltpu.CompilerParams(
            dimension_semantics=("arbitrary", "arbitrary"),
            vmem_limit_bytes=100 * 1024 * 1024),
    )(eob, act, xs, w1, b1, w2, b2, g3)


def _sc_gather(x, src):
    t, d = x.shape
    npad = src.shape[0]
    rows_pw = npad // _NW
    ch = 8
    nbuf = 4
    n_ch = rows_pw // ch
    n_grp = n_ch // nbuf
    mesh = plsc.VectorSubcoreMesh(core_axis_name="c", subcore_axis_name="s")

    @functools.partial(
        pl.kernel, mesh=mesh,
        out_type=jax.ShapeDtypeStruct((npad, d), jnp.float32),
        scratch_types=(
            [pltpu.VMEM((rows_pw,), jnp.int32)]
            + [pltpu.VMEM((ch, d), jnp.float32) for _ in range(nbuf)]
            + [pltpu.SemaphoreType.DMA for _ in range(2 * nbuf)]
        ),
    )
    def k(x_hbm, src_hbm, out_hbm, idx_v, *rest):
        bufs = rest[:nbuf]
        gsems = rest[nbuf:2 * nbuf]
        osems = rest[2 * nbuf:3 * nbuf]
        wid = lax.axis_index("s") * _NC + lax.axis_index("c")
        base = wid * rows_pw
        pltpu.sync_copy(src_hbm.at[pl.ds(base, rows_pw)], idx_v)
        for b in range(nbuf):
            pltpu.async_copy(x_hbm.at[idx_v.at[pl.ds(b * ch, ch)]],
                             bufs[b], gsems[b])

        def body(g, _):
            c0 = g * nbuf
            for b in range(nbuf):
                pltpu.make_async_copy(x_hbm.at[pl.ds(0, ch)], bufs[b],
                                      gsems[b]).wait()
                pltpu.async_copy(bufs[b],
                                 out_hbm.at[pl.ds(base + (c0 + b) * ch, ch)],
                                 osems[b])

            @pl.when(g + 1 < n_grp)
            def _():
                for b in range(nbuf):
                    pltpu.make_async_copy(
                        bufs[b], out_hbm.at[pl.ds(0, ch)], osems[b]).wait()
                    pltpu.async_copy(
                        x_hbm.at[idx_v.at[pl.ds((c0 + nbuf + b) * ch, ch)]],
                        bufs[b], gsems[b])
            return 0

        lax.fori_loop(0, n_grp, body, 0)
        for b in range(nbuf):
            pltpu.make_async_copy(bufs[b], out_hbm.at[pl.ds(0, ch)],
                                  osems[b]).wait()

    return k(x, src)


def _sc_combine(eo, pos0, pos1):
    npad, d = eo.shape
    t = pos0.shape[0]
    rows_pw = t // _NW
    ch = 8
    n_ch = rows_pw // ch
    n_half = n_ch // 2
    nvec = d // 16
    mesh = plsc.VectorSubcoreMesh(core_axis_name="c", subcore_axis_name="s")

    nbuf = 2
    n_grp = n_ch // nbuf

    @functools.partial(
        pl.kernel, mesh=mesh,
        out_type=jax.ShapeDtypeStruct((t, d), jnp.float32),
        scratch_types=(
            [pltpu.VMEM((rows_pw,), jnp.int32),
             pltpu.VMEM((rows_pw,), jnp.int32)]
            + [pltpu.VMEM((ch, d), jnp.float32) for _ in range(2 * nbuf)]
            + [pltpu.SemaphoreType.DMA for _ in range(2 * nbuf)]
        ),
    )
    def k(eo_hbm, p0_hbm, p1_hbm, out_hbm, p0_v, p1_v, *rest):
        abufs = rest[:nbuf]
        bbufs = rest[nbuf:2 * nbuf]
        gsems = rest[2 * nbuf:3 * nbuf]
        osems = rest[3 * nbuf:4 * nbuf]
        wid = lax.axis_index("s") * _NC + lax.axis_index("c")
        base = wid * rows_pw
        pltpu.sync_copy(p0_hbm.at[pl.ds(base, rows_pw)], p0_v)
        pltpu.sync_copy(p1_hbm.at[pl.ds(base, rows_pw)], p1_v)
        for b in range(nbuf):
            pltpu.async_copy(eo_hbm.at[p0_v.at[pl.ds(b * ch, ch)]],
                             abufs[b], gsems[b])
            pltpu.async_copy(eo_hbm.at[p1_v.at[pl.ds(b * ch, ch)]],
                             bbufs[b], gsems[b])

        def body(g, _):
            c0 = g * nbuf
            for b in range(nbuf):
                pltpu.make_async_copy(eo_hbm.at[pl.ds(0, ch)], abufs[b],
                                      gsems[b]).wait()
                pltpu.make_async_copy(eo_hbm.at[pl.ds(0, ch)], bbufs[b],
                                      gsems[b]).wait()

                def row(r, _, ab=abufs[b], bb=bbufs[b]):
                    for cc in range(nvec):
                        sl = pl.ds(cc * 16, 16)
                        ab[r, sl] = ab[r, sl] + bb[r, sl]
                    return 0

                lax.fori_loop(0, ch, row, 0)
                pltpu.async_copy(abufs[b],
                                 out_hbm.at[pl.ds(base + (c0 + b) * ch, ch)],
                                 osems[b])

            @pl.when(g + 1 < n_grp)
            def _():
                for b in range(nbuf):
                    c = c0 + nbuf + b
                    pltpu.make_async_copy(abufs[b], out_hbm.at[pl.ds(0, ch)],
                                          osems[b]).wait()
                    pltpu.async_copy(eo_hbm.at[p0_v.at[pl.ds(c * ch, ch)]],
                                     abufs[b], gsems[b])
                    pltpu.async_copy(eo_hbm.at[p1_v.at[pl.ds(c * ch, ch)]],
                                     bbufs[b], gsems[b])
            return 0

        lax.fori_loop(0, n_grp, body, 0)
        for b in range(nbuf):
            pltpu.make_async_copy(abufs[b], out_hbm.at[pl.ds(0, ch)],
                                  osems[b]).wait()

    return k(eo, pos0, pos1)


def kernel(x, wg, w1, b1, w2, b2, k):
    t, d = x.shape
    e = wg.shape[1]
    blk = 512 if t >= 4096 else 64
    nh = 8
    npad = 2 * t + e * blk
    nb = npad // blk

    x_bf = x.astype(jnp.bfloat16)
    x_view = lax.bitcast_convert_type(
        x_bf.reshape(t, d // 2, 2), jnp.float32)

    gvals, gidx, ssum, msum = _gate(x_bf, wg)

    eye = jnp.eye(e, dtype=jnp.int32)
    oh0 = eye[gidx[:, 0]]
    oh1 = eye[gidx[:, 1]]
    pc0 = jnp.cumsum(oh0, axis=0) - oh0
    pc1 = jnp.cumsum(oh1, axis=0) - oh1
    c0 = pc0[-1] + oh0[-1]
    c1 = pc1[-1] + oh1[-1]
    counts = c0 + c1
    pc = ((counts + blk - 1) // blk) * blk
    base = jnp.concatenate([jnp.zeros((1,), jnp.int32),
                            jnp.cumsum(pc)])[:e]
    dest0 = (base + pc0)[jnp.arange(t), gidx[:, 0]]
    dest1 = (base + c0 + pc1)[jnp.arange(t), gidx[:, 1]]
    toki = jnp.arange(t, dtype=jnp.int32)
    fill = (jnp.arange(npad, dtype=jnp.int32) * 37) % t
    src = fill.at[dest0].set(toki).at[dest1].set(toki)
    grow = jnp.zeros((npad,), jnp.float32)
    grow = grow.at[dest0].set(gvals[:, 0]).at[dest1].set(gvals[:, 1])
    pos = jnp.stack([dest0, dest1], axis=1)
    nbe = pc // blk
    eob = jnp.repeat(jnp.arange(e, dtype=jnp.int32), nbe,
                     total_repeat_length=nb)
    used = jnp.sum(nbe)
    act = (jnp.arange(nb, dtype=jnp.int32) < used).astype(jnp.int32)

    xs_view = _sc_gather(x_view, src)
    xs = lax.bitcast_convert_type(xs_view, jnp.bfloat16).reshape(npad, d)

    g3 = grow.reshape(nb, 1, blk)
    eo = _ffn(eob, act, xs, w1, b1, w2, b2, g3, blk, nh)

    out = _sc_combine(eo, pos[:, 0], pos[:, 1])

    l_aux = e * jnp.sum((ssum[0] / t) * (msum[0] / t))
    return out, l_aux


# speedup vs baseline: 1.5083x; 1.5083x over previous
import functools

import jax
import jax.numpy as jnp
from jax import lax
from jax.experimental import pallas as pl
from jax.experimental.pallas import tpu as pltpu
from jax.experimental.pallas import tpu_sc as plsc

_NC = 2
_NS = 16
_NW = _NC * _NS


def _gate_body(x_ref, wg_ref, gv_ref, gi_ref, ss_ref, ms_ref):
    i = pl.program_id(0)
    lg = lax.dot_general(x_ref[...].astype(jnp.bfloat16),
                         wg_ref[...].astype(jnp.bfloat16),
                         (((1,), (0,)), ((), ())),
                         preferred_element_type=jnp.float32)
    m = jnp.max(lg, axis=-1, keepdims=True)
    ex = jnp.exp(lg - m)
    p = ex / jnp.sum(ex, axis=-1, keepdims=True)
    e = p.shape[-1]
    i8 = lax.broadcasted_iota(jnp.int32, p.shape, 1)
    v1 = jnp.max(p, axis=-1, keepdims=True)
    a1 = jnp.min(jnp.where(p >= v1, i8, e), axis=-1)
    mask1 = i8 == a1[:, None]
    p2 = jnp.where(mask1, -1.0, p)
    v2 = jnp.max(p2, axis=-1, keepdims=True)
    a2 = jnp.min(jnp.where(p2 >= v2, i8, e), axis=-1)
    gv_ref[...] = jnp.concatenate([v1, v2], axis=-1)
    gi_ref[...] = jnp.concatenate([a1[:, None], a2[:, None]], axis=-1)

    @pl.when(i == 0)
    def _():
        ss_ref[...] = jnp.zeros_like(ss_ref)
        ms_ref[...] = jnp.zeros_like(ms_ref)

    ss_ref[...] += jnp.sum(p, axis=0, keepdims=True)
    ms_ref[...] += jnp.sum(mask1.astype(jnp.float32), axis=0, keepdims=True)


def _gate(x, wg):
    t, d = x.shape
    e = wg.shape[1]
    tb = min(512, t)
    grid = (t // tb,)
    return pl.pallas_call(
        _gate_body,
        grid=grid,
        in_specs=[
            pl.BlockSpec((tb, d), lambda i: (i, 0)),
            pl.BlockSpec((d, e), lambda i: (0, 0)),
        ],
        out_specs=[
            pl.BlockSpec((tb, 2), lambda i: (i, 0)),
            pl.BlockSpec((tb, 2), lambda i: (i, 0)),
            pl.BlockSpec((1, e), lambda i: (0, 0)),
            pl.BlockSpec((1, e), lambda i: (0, 0)),
        ],
        out_shape=[
            jax.ShapeDtypeStruct((t, 2), jnp.float32),
            jax.ShapeDtypeStruct((t, 2), jnp.int32),
            jax.ShapeDtypeStruct((1, e), jnp.float32),
            jax.ShapeDtypeStruct((1, e), jnp.float32),
        ],
        compiler_params=pltpu.CompilerParams(
            dimension_semantics=("arbitrary",)),
    )(x, wg)


def _ffn_body(nh, eob_ref, act_ref, xs_ref, w1_ref, b1_ref, w2_ref, b2_ref,
              g_ref, eo_ref):
    rb = pl.program_id(0)
    hb = pl.program_id(1)
    active = act_ref[rb] > 0

    @pl.when(active)
    def _():
        xb = xs_ref[...].astype(jnp.bfloat16)
        w1b = w1_ref[0].astype(jnp.bfloat16)
        h = lax.dot_general(xb, w1b, (((1,), (0,)), ((), ())),
                            preferred_element_type=jnp.float32)
        h = jnp.maximum(h + b1_ref[0], 0.0).astype(jnp.bfloat16)
        pt = lax.dot_general(h, w2_ref[0].astype(jnp.bfloat16),
                             (((1,), (0,)), ((), ())),
                             preferred_element_type=jnp.float32)

        @pl.when(hb == 0)
        def _():
            eo_ref[...] = pt

        @pl.when(hb > 0)
        def _():
            eo_ref[...] += pt

        @pl.when(hb == nh - 1)
        def _():
            g = g_ref[0, 0, :][:, None]
            eo_ref[...] = g * (eo_ref[...] + b2_ref[0])


def _ffn(eob, act, xs, w1, b1, w2, b2, g3, blk, nh):
    npad, d = xs.shape
    e, _, h = w1.shape
    hblk = h // nh
    nb = npad // blk
    grid_spec = pltpu.PrefetchScalarGridSpec(
        num_scalar_prefetch=2,
        grid=(nb, nh),
        in_specs=[
            pl.BlockSpec((blk, d), lambda rb, hb, eob, act: (rb, 0)),
            pl.BlockSpec((1, d, hblk),
                         lambda rb, hb, eob, act: (eob[rb], 0, hb * act[rb])),
            pl.BlockSpec((1, 1, hblk),
                         lambda rb, hb, eob, act: (eob[rb], 0, hb * act[rb])),
            pl.BlockSpec((1, hblk, d),
                         lambda rb, hb, eob, act: (eob[rb], hb * act[rb], 0)),
            pl.BlockSpec((1, 1, d),
                         lambda rb, hb, eob, act: (eob[rb], 0, 0)),
            pl.BlockSpec((1, 1, blk), lambda rb, hb, eob, act: (rb, 0, 0)),
        ],
        out_specs=pl.BlockSpec((blk, d), lambda rb, hb, eob, act: (rb, 0)),
    )
    return pl.pallas_call(
        functools.partial(_ffn_body, nh),
        grid_spec=grid_spec,
        out_shape=jax.ShapeDtypeStruct((npad, d), jnp.float32),
        compiler_params=pltpu.CompilerParams(
            dimension_semantics=("arbitrary", "arbitrary"),
            vmem_limit_bytes=100 * 1024 * 1024),
    )(eob, act, xs, w1, b1, w2, b2, g3)


def _sc_gather(x, src):
    t, d = x.shape
    npad = src.shape[0]
    rows_pw = npad // _NW
    ch = 8
    nbuf = 4
    n_ch = rows_pw // ch
    n_grp = n_ch // nbuf
    mesh = plsc.VectorSubcoreMesh(core_axis_name="c", subcore_axis_name="s")

    @functools.partial(
        pl.kernel, mesh=mesh,
        out_type=jax.ShapeDtypeStruct((npad, d), jnp.float32),
        scratch_types=(
            [pltpu.VMEM((rows_pw,), jnp.int32)]
            + [pltpu.VMEM((ch, d), jnp.float32) for _ in range(nbuf)]
            + [pltpu.SemaphoreType.DMA for _ in range(2 * nbuf)]
        ),
    )
    def k(x_hbm, src_hbm, out_hbm, idx_v, *rest):
        bufs = rest[:nbuf]
        gsems = rest[nbuf:2 * nbuf]
        osems = rest[2 * nbuf:3 * nbuf]
        wid = lax.axis_index("s") * _NC + lax.axis_index("c")
        base = wid * rows_pw
        pltpu.sync_copy(src_hbm.at[pl.ds(base, rows_pw)], idx_v)
        for b in range(nbuf):
            pltpu.async_copy(x_hbm.at[idx_v.at[pl.ds(b * ch, ch)]],
                             bufs[b], gsems[b])

        def body(g, _):
            c0 = g * nbuf
            for b in range(nbuf):
                pltpu.make_async_copy(x_hbm.at[pl.ds(0, ch)], bufs[b],
                                      gsems[b]).wait()
                pltpu.async_copy(bufs[b],
                                 out_hbm.at[pl.ds(base + (c0 + b) * ch, ch)],
                                 osems[b])

            @pl.when(g + 1 < n_grp)
            def _():
                for b in range(nbuf):
                    pltpu.make_async_copy(
                        bufs[b], out_hbm.at[pl.ds(0, ch)], osems[b]).wait()
                    pltpu.async_copy(
                        x_hbm.at[idx_v.at[pl.ds((c0 + nbuf + b) * ch, ch)]],
                        bufs[b], gsems[b])
            return 0

        lax.fori_loop(0, n_grp, body, 0)
        for b in range(nbuf):
            pltpu.make_async_copy(bufs[b], out_hbm.at[pl.ds(0, ch)],
                                  osems[b]).wait()

    return k(x, src)


def _sc_combine(eo, pos0, pos1):
    npad, d = eo.shape
    t = pos0.shape[0]
    rows_pw = t // _NW
    ch = 8
    n_ch = rows_pw // ch
    n_half = n_ch // 2
    nvec = d // 16
    mesh = plsc.VectorSubcoreMesh(core_axis_name="c", subcore_axis_name="s")

    nbuf = 2
    n_grp = n_ch // nbuf

    @functools.partial(
        pl.kernel, mesh=mesh,
        out_type=jax.ShapeDtypeStruct((t, d), jnp.float32),
        scratch_types=(
            [pltpu.VMEM((rows_pw,), jnp.int32),
             pltpu.VMEM((rows_pw,), jnp.int32)]
            + [pltpu.VMEM((ch, d), jnp.float32) for _ in range(2 * nbuf)]
            + [pltpu.SemaphoreType.DMA for _ in range(2 * nbuf)]
        ),
    )
    def k(eo_hbm, p0_hbm, p1_hbm, out_hbm, p0_v, p1_v, *rest):
        abufs = rest[:nbuf]
        bbufs = rest[nbuf:2 * nbuf]
        gsems = rest[2 * nbuf:3 * nbuf]
        osems = rest[3 * nbuf:4 * nbuf]
        wid = lax.axis_index("s") * _NC + lax.axis_index("c")
        base = wid * rows_pw
        pltpu.sync_copy(p0_hbm.at[pl.ds(base, rows_pw)], p0_v)
        pltpu.sync_copy(p1_hbm.at[pl.ds(base, rows_pw)], p1_v)
        for b in range(nbuf):
            pltpu.async_copy(eo_hbm.at[p0_v.at[pl.ds(b * ch, ch)]],
                             abufs[b], gsems[b])
            pltpu.async_copy(eo_hbm.at[p1_v.at[pl.ds(b * ch, ch)]],
                             bbufs[b], gsems[b])

        def body(g, _):
            c0 = g * nbuf
            for b in range(nbuf):
                pltpu.make_async_copy(eo_hbm.at[pl.ds(0, ch)], abufs[b],
                                      gsems[b]).wait()
                pltpu.make_async_copy(eo_hbm.at[pl.ds(0, ch)], bbufs[b],
                                      gsems[b]).wait()

                def row(r, _, ab=abufs[b], bb=bbufs[b]):
                    for cc in range(nvec):
                        sl = pl.ds(cc * 16, 16)
                        ab[r, sl] = ab[r, sl] + bb[r, sl]
                    return 0

                lax.fori_loop(0, ch, row, 0)
                pltpu.async_copy(abufs[b],
                                 out_hbm.at[pl.ds(base + (c0 + b) * ch, ch)],
                                 osems[b])

            @pl.when(g + 1 < n_grp)
            def _():
                for b in range(nbuf):
                    c = c0 + nbuf + b
                    pltpu.make_async_copy(abufs[b], out_hbm.at[pl.ds(0, ch)],
                                          osems[b]).wait()
                    pltpu.async_copy(eo_hbm.at[p0_v.at[pl.ds(c * ch, ch)]],
                                     abufs[b], gsems[b])
                    pltpu.async_copy(eo_hbm.at[p1_v.at[pl.ds(c * ch, ch)]],
                                     bbufs[b], gsems[b])
            return 0

        lax.fori_loop(0, n_grp, body, 0)
        for b in range(nbuf):
            pltpu.make_async_copy(abufs[b], out_hbm.at[pl.ds(0, ch)],
                                  osems[b]).wait()

    return k(eo, pos0, pos1)


def kernel(x, wg, w1, b1, w2, b2, k):
    t, d = x.shape
    e = wg.shape[1]
    blk = 512 if t >= 4096 else 64
    nh = 8
    npad = 2 * t + e * blk
    nb = npad // blk

    gvals, gidx, ssum, msum = _gate(x, wg)

    eye = jnp.eye(e, dtype=jnp.int32)
    oh0 = eye[gidx[:, 0]]
    oh1 = eye[gidx[:, 1]]
    pc0 = jnp.cumsum(oh0, axis=0) - oh0
    pc1 = jnp.cumsum(oh1, axis=0) - oh1
    c0 = pc0[-1] + oh0[-1]
    c1 = pc1[-1] + oh1[-1]
    counts = c0 + c1
    pc = ((counts + blk - 1) // blk) * blk
    base = jnp.concatenate([jnp.zeros((1,), jnp.int32),
                            jnp.cumsum(pc)])[:e]
    dest0 = (base + pc0)[jnp.arange(t), gidx[:, 0]]
    dest1 = (base + c0 + pc1)[jnp.arange(t), gidx[:, 1]]
    toki = jnp.arange(t, dtype=jnp.int32)
    fill = (jnp.arange(npad, dtype=jnp.int32) * 37) % t
    src = fill.at[dest0].set(toki).at[dest1].set(toki)
    grow = jnp.zeros((npad,), jnp.float32)
    grow = grow.at[dest0].set(gvals[:, 0]).at[dest1].set(gvals[:, 1])
    pos = jnp.stack([dest0, dest1], axis=1)
    nbe = pc // blk
    eob = jnp.repeat(jnp.arange(e, dtype=jnp.int32), nbe,
                     total_repeat_length=nb)
    used = jnp.sum(nbe)
    act = (jnp.arange(nb, dtype=jnp.int32) < used).astype(jnp.int32)

    xs = _sc_gather(x, src)

    g3 = grow.reshape(nb, 1, blk)
    eo = _ffn(eob, act, xs, w1, b1, w2, b2, g3, blk, nh)

    out = _sc_combine(eo, pos[:, 0], pos[:, 1])

    l_aux = e * jnp.sum((ssum[0] / t) * (msum[0] / t))
    return out, l_aux
